# baseline (speedup 1.0000x reference)
import functools

import jax
import jax.numpy as jnp
from jax import lax
from jax.experimental import pallas as pl
from jax.experimental.pallas import tpu as pltpu
from jax.experimental.pallas import tpu_sc as plsc

_INFO = plsc.get_sparse_core_info()
_NC = _INFO.num_cores
_NS = _INFO.num_subcores
_NW = _NC * _NS
_L = _INFO.num_lanes

_CHUNK = 2048
_HALF = _CHUNK // 2
_IBUF = 3
_PITCH = _HALF + 1


@functools.partial(jax.jit, static_argnums=(2, 3))
def _gather_t(idx_t, table, batch, hist):
    D = table.shape[1]
    n_hgrp = 8
    n_bq = _NW // n_hgrp
    h_per_grp = hist // n_hgrp
    b_per_q = batch // n_bq
    nchunks = h_per_grp * (b_per_q // _CHUNK)
    mesh = plsc.VectorSubcoreMesh(core_axis_name="c", subcore_axis_name="s")

    @functools.partial(
        pl.kernel,
        mesh=mesh,
        out_type=jax.ShapeDtypeStruct((hist, D, batch), jnp.float32),
        scratch_types=[
            pltpu.VMEM((_IBUF, _CHUNK), jnp.int32),
            pltpu.VMEM((2, _CHUNK, D), jnp.float32),
            pltpu.VMEM((2, D, _PITCH), jnp.float32),
            pltpu.SemaphoreType.DMA((_IBUF,)),
            pltpu.SemaphoreType.DMA((2,)),
            pltpu.SemaphoreType.DMA((2,)),
        ],
        compiler_params=pltpu.CompilerParams(
            use_tc_tiling_on_sc=False, needs_layout_passes=False),
    )
    def k(idx_hbm, table_hbm, out_hbm, idx_v, rows_v, cols_v,
          isem, gsem, osem):
        wid = lax.axis_index("s") * _NC + lax.axis_index("c")
        h_base = (wid // n_bq) * h_per_grp
        b_base = (wid % n_bq) * b_per_q
        last = nchunks - 1

        def idx_cp(c):
            s = lax.rem(c, _IBUF)
            off = (h_base + c // 2) * batch + b_base + lax.rem(c, 2) * _CHUNK
            return pltpu.make_async_copy(
                idx_hbm.at[pl.ds(off, _CHUNK)], idx_v.at[s], isem.at[s])

        def gather(c):
            s = lax.rem(c, 2)
            return pltpu.make_async_copy(
                table_hbm.at[idx_v.at[lax.rem(c, _IBUF)]], rows_v.at[s],
                gsem.at[s])

        def out(c, sub):
            h = h_base + c // 2
            b0 = b_base + lax.rem(c, 2) * _CHUNK + sub * _HALF
            return pltpu.make_async_copy(
                cols_v.at[sub, :, pl.ds(0, _HALF)],
                out_hbm.at[h, :, pl.ds(b0, _HALF)], osem.at[sub])

        def out_start(c, sub):
            out(c, sub).start()

        def out_wait(c, sub):
            out(c, sub).wait()

        lane = lax.iota(jnp.int32, _L)
        zero = jnp.zeros((_L,), jnp.int32)
        _UNROLL = 8

        def transpose(rs, sub):
            base = sub * _HALF

            def body(u, ii):
                i0 = base + u * _UNROLL
                for du in range(_UNROLL):
                    v = rows_v[rs, i0 + du, :]
                    plsc.store_scatter(cols_v.at[sub], [lane, ii], v)
                    ii = ii + 1
                return ii
            lax.fori_loop(0, _HALF // _UNROLL, body, zero)

        def finish(c, osem_wait):
            rs = lax.rem(c, 2)
            for sub in range(2):
                if osem_wait:
                    out_wait(c - 1, sub)
                transpose(rs, sub)
                out_start(c, sub)

        for c in range(_IBUF):
            idx_cp(c).start()
        idx_cp(0).wait()
        gather(0).start()
        idx_cp(1).wait()
        gather(1).start()
        gather(0).wait()
        idx_cp(_IBUF).start()
        finish(0, False)

        def step(c, carry):
            idx_cp(c).wait()
            gather(c).start()
            gather(c - 1).wait()
            @pl.when(c + _IBUF - 1 <= last)
            def _():
                idx_cp(c + _IBUF - 1).start()
            finish(c - 1, True)
            return carry

        lax.fori_loop(2, nchunks, step, 0)

        gather(last).wait()
        finish(last, True)
        for sub in range(2):
            out_wait(last, sub)

    return k(idx_t, table)


def kernel(input_ids, table):
    Bt, H = input_ids.shape
    D = table.shape[1]
    idx_t = input_ids.T.reshape(-1).astype(jnp.int32)
    out_t = _gather_t(idx_t, table, Bt, H)
    return out_t.transpose(2, 0, 1)


# speedup vs baseline: 1.0387x; 1.0387x over previous
import functools

import jax
import jax.numpy as jnp
from jax import lax
from jax.experimental import pallas as pl
from jax.experimental.pallas import tpu as pltpu
from jax.experimental.pallas import tpu_sc as plsc

_INFO = plsc.get_sparse_core_info()
_NC = _INFO.num_cores
_NS = _INFO.num_subcores
_NW = _NC * _NS
_L = _INFO.num_lanes

_CHUNK = 2048
_HALF = _CHUNK // 2
_IBUF = 3
_PITCH = _HALF + 1


@functools.partial(jax.jit, static_argnums=(2, 3))
def _gather_t(idx_t, table, batch, hist):
    D = table.shape[1]
    n_hgrp = 8
    n_bq = _NW // n_hgrp
    h_per_grp = hist // n_hgrp
    b_per_q = batch // n_bq
    nchunks = h_per_grp * (b_per_q // _CHUNK)
    mesh = plsc.VectorSubcoreMesh(core_axis_name="c", subcore_axis_name="s")

    @functools.partial(
        pl.kernel,
        mesh=mesh,
        out_type=jax.ShapeDtypeStruct((hist, D, batch), jnp.float32),
        scratch_types=[
            pltpu.VMEM((_IBUF, _CHUNK), jnp.int32),
            pltpu.VMEM((2, _CHUNK, D), jnp.float32),
            pltpu.VMEM((2, D, _PITCH), jnp.float32),
            pltpu.SemaphoreType.DMA((_IBUF,)),
            pltpu.SemaphoreType.DMA((2,)),
            pltpu.SemaphoreType.DMA((2,)),
        ],
        compiler_params=pltpu.CompilerParams(
            use_tc_tiling_on_sc=False, needs_layout_passes=False),
    )
    def k(idx_hbm, table_hbm, out_hbm, idx_v, rows_v, cols_v,
          isem, gsem, osem):
        wid = lax.axis_index("s") * _NC + lax.axis_index("c")
        h_base = (wid // n_bq) * h_per_grp
        b_base = (wid % n_bq) * b_per_q
        last = nchunks - 1

        def idx_cp(c):
            s = lax.rem(c, _IBUF)
            off = (h_base + c // 2) * batch + b_base + lax.rem(c, 2) * _CHUNK
            return pltpu.make_async_copy(
                idx_hbm.at[pl.ds(off, _CHUNK)], idx_v.at[s], isem.at[s])

        def gather(c):
            s = lax.rem(c, 2)
            return pltpu.make_async_copy(
                table_hbm.at[idx_v.at[lax.rem(c, _IBUF)]], rows_v.at[s],
                gsem.at[s])

        def out(c, sub):
            h = h_base + c // 2
            b0 = b_base + lax.rem(c, 2) * _CHUNK + sub * _HALF
            return pltpu.make_async_copy(
                cols_v.at[sub, :, pl.ds(0, _HALF)],
                out_hbm.at[h, :, pl.ds(b0, _HALF)], osem.at[sub])

        def out_start(c, sub):
            out(c, sub).start()

        def out_wait(c, sub):
            out(c, sub).wait()

        lane = lax.iota(jnp.int32, _L)
        _UNROLL = 8

        def transpose(rs, sub):
            base = sub * _HALF

            def body(u, carry):
                i0 = u * _UNROLL
                for du in range(_UNROLL):
                    v = rows_v[rs, base + i0 + du, :]
                    ii = jnp.full((_L,), i0 + du, jnp.int32)
                    plsc.store_scatter(cols_v.at[sub], [lane, ii], v)
                return carry
            lax.fori_loop(0, _HALF // _UNROLL, body, 0)

        def finish(c, osem_wait):
            rs = lax.rem(c, 2)
            for sub in range(2):
                if osem_wait:
                    out_wait(c - 1, sub)
                transpose(rs, sub)
                out_start(c, sub)

        for c in range(_IBUF):
            idx_cp(c).start()
        idx_cp(0).wait()
        gather(0).start()
        idx_cp(1).wait()
        gather(1).start()
        gather(0).wait()
        idx_cp(_IBUF).start()
        finish(0, False)

        def step(c, carry):
            idx_cp(c).wait()
            gather(c).start()
            gather(c - 1).wait()
            @pl.when(c + _IBUF - 1 <= last)
            def _():
                idx_cp(c + _IBUF - 1).start()
            finish(c - 1, True)
            return carry

        lax.fori_loop(2, nchunks, step, 0)

        gather(last).wait()
        finish(last, True)
        for sub in range(2):
            out_wait(last, sub)

    return k(idx_t, table)


def kernel(input_ids, table):
    Bt, H = input_ids.shape
    D = table.shape[1]
    idx_t = input_ids.T.reshape(-1).astype(jnp.int32)
    out_t = _gather_t(idx_t, table, Bt, H)
    return out_t.transpose(2, 0, 1)
